# baseline (speedup 1.0000x reference)
import functools

import jax
import jax.numpy as jnp
from jax import lax
from jax.experimental import pallas as pl
from jax.experimental.pallas import tpu as pltpu
from jax.experimental.pallas import tpu_sc as plsc

N = 10000
E = 320000
IN_FEATS = 128
HID = 16
OUT_FEATS = 64
H1 = 8

_BLK = 400

_NSUB = 16
_HP = 16


def _dense_body(x_ref, w_ref, o1_ref, o2_ref, o3_ref, o4_ref):
    z = jnp.dot(x_ref[...], w_ref[...], preferred_element_type=jnp.float32)
    cf = o1_ref.shape[1]
    o1_ref[...] = z[:, :cf]
    o2_ref[...] = z[:, cf:2 * cf]
    o3_ref[...] = z[:, 2 * cf:2 * cf + _HP]
    o4_ref[...] = z[:, 2 * cf + _HP:]


def _dense(x, wbig, cf):
    n, k = x.shape
    m = wbig.shape[1]
    return pl.pallas_call(
        _dense_body,
        grid=(n // _BLK,),
        in_specs=[
            pl.BlockSpec((_BLK, k), lambda i: (i, 0)),
            pl.BlockSpec((k, m), lambda i: (0, 0)),
        ],
        out_specs=[
            pl.BlockSpec((_BLK, cf), lambda i: (i, 0)),
            pl.BlockSpec((_BLK, cf), lambda i: (i, 0)),
            pl.BlockSpec((_BLK, _HP), lambda i: (i, 0)),
            pl.BlockSpec((_BLK, _HP), lambda i: (i, 0)),
        ],
        out_shape=[
            jax.ShapeDtypeStruct((n, cf), jnp.float32),
            jax.ShapeDtypeStruct((n, cf), jnp.float32),
            jax.ShapeDtypeStruct((n, _HP), jnp.float32),
            jax.ShapeDtypeStruct((n, _HP), jnp.float32),
        ],
    )(x, wbig)


def _combine1_body(pa_ref, pb_ref, e0_ref, e1_ref, b_ref, w_ref,
                   o1_ref, o2_ref, o3_ref, o4_ref):
    pa = pa_ref[...]
    pb = pb_ref[...]
    half = IN_FEATS // 2
    num = jnp.concatenate([pa[:, :half], pb[:, :half]], axis=1)
    denx = (jnp.dot(pa[:, half:], e0_ref[...],
                    preferred_element_type=jnp.float32)
            + jnp.dot(pb[:, half:], e1_ref[...],
                      preferred_element_type=jnp.float32))
    h = num / (denx + 1e-9) + b_ref[...]
    h = jnp.maximum(h, 0.0)
    z = jnp.dot(h, w_ref[...], preferred_element_type=jnp.float32)
    cf = o1_ref.shape[1]
    o1_ref[...] = z[:, :cf]
    o2_ref[...] = z[:, cf:2 * cf]
    o3_ref[...] = z[:, 2 * cf:2 * cf + _HP]
    o4_ref[...] = z[:, 2 * cf + _HP:]


def _combine1(pa, pb, e0, e1, b1, w2big, cf):
    n = pa.shape[0]
    cc = pa.shape[1]
    m = w2big.shape[1]
    blk = pl.BlockSpec((_BLK, cc), lambda i: (i, 0))
    return pl.pallas_call(
        _combine1_body,
        grid=(n // _BLK,),
        in_specs=[
            blk, blk,
            pl.BlockSpec((_HP, IN_FEATS), lambda i: (0, 0)),
            pl.BlockSpec((_HP, IN_FEATS), lambda i: (0, 0)),
            pl.BlockSpec((1, IN_FEATS), lambda i: (0, 0)),
            pl.BlockSpec((IN_FEATS, m), lambda i: (0, 0)),
        ],
        out_specs=[
            pl.BlockSpec((_BLK, cf), lambda i: (i, 0)),
            pl.BlockSpec((_BLK, cf), lambda i: (i, 0)),
            pl.BlockSpec((_BLK, _HP), lambda i: (i, 0)),
            pl.BlockSpec((_BLK, _HP), lambda i: (i, 0)),
        ],
        out_shape=[
            jax.ShapeDtypeStruct((n, cf), jnp.float32),
            jax.ShapeDtypeStruct((n, cf), jnp.float32),
            jax.ShapeDtypeStruct((n, _HP), jnp.float32),
            jax.ShapeDtypeStruct((n, _HP), jnp.float32),
        ],
    )(pa, pb, e0, e1, b1, w2big)


def _combine2_body(pa_ref, pb_ref, b_ref, o_ref):
    pa = pa_ref[...]
    pb = pb_ref[...]
    half = OUT_FEATS // 2
    num = jnp.concatenate([pa[:, :half], pb[:, :half]], axis=1)
    den = pa[:, half:half + 1]
    o_ref[...] = num / (den + 1e-9) + b_ref[...]


def _combine2(pa, pb, b2):
    n = pa.shape[0]
    cc = pa.shape[1]
    blk = pl.BlockSpec((_BLK, cc), lambda i: (i, 0))
    return pl.pallas_call(
        _combine2_body,
        grid=(n // _BLK,),
        in_specs=[
            blk, blk,
            pl.BlockSpec((1, OUT_FEATS), lambda i: (0, 0)),
        ],
        out_specs=pl.BlockSpec((_BLK, OUT_FEATS), lambda i: (i, 0)),
        out_shape=jax.ShapeDtypeStruct((n, OUT_FEATS), jnp.float32),
    )(pa, pb, b2)


@functools.lru_cache(maxsize=None)
def _make_edge_sc(C, H, B):
    D = C // H
    CC = C + _HP
    NB = E // B
    NBT = NB // _NSUB
    NEXTRA = NB % _NSUB
    CNT_MAX = NBT + (1 if NEXTRA else 0)
    ROWS_T = (N // _NSUB) // 8 * 8
    ROWS_REM = N - ROWS_T * _NSUB

    mesh = plsc.VectorSubcoreMesh(core_axis_name="c", subcore_axis_name="s")

    def buf_pair(shape, dtype):
        return [pltpu.MemorySpace.VMEM(shape, dtype) for _ in range(2)]

    @functools.partial(
        pl.kernel,
        out_type=jax.ShapeDtypeStruct((2, N, CC), jnp.float32),
        mesh=mesh,
        compiler_params=pltpu.CompilerParams(use_tc_tiling_on_sc=False),
        scratch_types=[
            pltpu.MemorySpace.VMEM_SHARED((N, CC), jnp.float32),
            buf_pair((B,), jnp.int32),
            buf_pair((1, B), jnp.int32),
            buf_pair((1, B), jnp.int32),
            buf_pair((B, _HP), jnp.float32),
            buf_pair((B, CC), jnp.float32),
            buf_pair((B, CC), jnp.float32),
            [pltpu.SemaphoreType.DMA for _ in range(2)],
            [pltpu.SemaphoreType.DMA for _ in range(2)],
            [pltpu.SemaphoreType.DMA for _ in range(2)],
        ],
    )
    def edge_kernel(fel0, fel1, er0, er1, src, dst, acc_out,
                    acc, sidx, didx, didxs, erb, fbg, mb,
                    isem, gsem, ssem):
        c = lax.axis_index("c")
        s = lax.axis_index("s")
        zeros16 = jnp.zeros((16,), jnp.float32)

        nmine = NBT + jnp.where(s < NEXTRA, 1, 0) if NEXTRA else NBT
        start_blk = NBT * s + jnp.minimum(s, NEXTRA)

        def issue_idx(b, k):
            off = (start_blk + k) * B
            pltpu.async_copy(src.at[pl.ds(off, B)], sidx[b], isem[b])
            pltpu.async_copy(dst.at[pl.ds(off, B)], didx[b].at[0], isem[b])

        def wait_idx(b, k):
            off = (start_blk + k) * B
            pltpu.make_async_copy(src.at[pl.ds(off, B)], sidx[b],
                                  isem[b]).wait()
            pltpu.make_async_copy(dst.at[pl.ds(off, B)], didx[b].at[0],
                                  isem[b]).wait()

        def issue_gathers(b):
            @pl.when(c == 0)
            def _g0():
                pltpu.async_copy(fel0.at[sidx[b]], fbg[b], gsem[b])
                pltpu.async_copy(er0.at[didx[b].at[0]], erb[b], gsem[b])

            @pl.when(c == 1)
            def _g1():
                pltpu.async_copy(fel1.at[sidx[b]], fbg[b], gsem[b])
                pltpu.async_copy(er1.at[didx[b].at[0]], erb[b], gsem[b])

        def wait_gathers(b):
            pltpu.make_async_copy(fel0.at[sidx[b]], fbg[b], gsem[b]).wait()
            pltpu.make_async_copy(er0.at[didx[b].at[0]], erb[b],
                                  gsem[b]).wait()

        def issue_scatter(b):
            pltpu.async_copy(mb[b], acc.at[didxs[b].at[0]], ssem[b], add=True)

        def wait_scatter(b):
            pltpu.make_async_copy(mb[b], acc.at[didxs[b].at[0]],
                                  ssem[b]).wait()

        def compute(b):
            @pl.loop(0, B, unroll=4)
            def _m(i):
                ev = fbg[b][i, pl.ds(C, 16)] + erb[b][i, :]
                wrow = jnp.exp(jnp.maximum(ev, 0.2 * ev))
                mb[b][i, pl.ds(C, 16)] = wrow
                for h in range(H):
                    wv = jnp.broadcast_to(wrow[h], (16,))
                    for t in range(D // 16):
                        col = h * D + t * 16
                        mb[b][i, pl.ds(col, 16)] = fbg[b][i, pl.ds(col, 16)] * wv

        for b in range(2):
            @pl.when(b < nmine)
            def _prime():
                issue_idx(b, b)
                wait_idx(b, b)
                issue_gathers(b)

        @pl.loop(0, B * CC // 16)
        def _zf(k):
            i = k // (CC // 16)
            j = k % (CC // 16)
            mb[0][i, pl.ds(j * 16, 16)] = zeros16

        r0 = s * ROWS_T
        nfull = ROWS_T // B
        rem = ROWS_T - nfull * B
        for q in range(nfull):
            pltpu.sync_copy(mb[0], acc.at[pl.ds(r0 + q * B, B)])
        if rem:
            pltpu.sync_copy(mb[0].at[pl.ds(0, rem)],
                            acc.at[pl.ds(r0 + nfull * B, rem)])

        @pl.when(s == _NSUB - 1)
        def _ztail():
            base = ROWS_T * _NSUB
            pltpu.sync_copy(mb[0].at[pl.ds(0, ROWS_REM)],
                            acc.at[pl.ds(base, ROWS_REM)])

        plsc.subcore_barrier()

        @pl.loop(0, (CNT_MAX + 1) // 2)
        def _blk(p):
            for b in range(2):
                k = p * 2 + b

                @pl.when(k < nmine)
                def _do():
                    wait_gathers(b)

                    @pl.loop(0, B // 16)
                    def _ci(q):
                        didxs[b][0, pl.ds(q * 16, 16)] = \
                            didx[b][0, pl.ds(q * 16, 16)]

                    @pl.when(k + 2 < nmine)
                    def _ni():
                        issue_idx(b, k + 2)

                    @pl.when(k >= 2)
                    def _ws():
                        wait_scatter(b)

                    compute(b)
                    issue_scatter(b)

                    @pl.when(k + 2 < nmine)
                    def _nx():
                        wait_idx(b, k + 2)
                        issue_gathers(b)

        for b in range(2):
            @pl.when(b < nmine)
            def _drain():
                wait_scatter(b)

        plsc.subcore_barrier()
        pltpu.sync_copy(acc.at[pl.ds(r0, ROWS_T)],
                        acc_out.at[c, pl.ds(r0, ROWS_T)])

        @pl.when(s == _NSUB - 1)
        def _ftail():
            base = ROWS_T * _NSUB
            pltpu.sync_copy(acc.at[pl.ds(base, ROWS_REM)],
                            acc_out.at[c, pl.ds(base, ROWS_REM)])

    return edge_kernel


def kernel(x, edge_index, W1, attn_l1, attn_r1, b1, W2, attn_l2, attn_r2, b2):
    src = edge_index[0]
    dst = edge_index[1]
    f32 = jnp.float32

    al1 = jax.scipy.linalg.block_diag(
        *[attn_l1[h][:, None] for h in range(H1)])
    ar1 = jax.scipy.linalg.block_diag(
        *[attn_r1[h][:, None] for h in range(H1)])
    hh = H1 // 2
    zp = jnp.zeros((IN_FEATS, _HP - hh), f32)
    wbig1 = jnp.concatenate(
        [W1[:, :64], W1 @ al1[:, :hh], zp,
         W1[:, 64:], W1 @ al1[:, hh:], zp,
         W1 @ ar1[:, :hh], zp,
         W1 @ ar1[:, hh:], zp], axis=1)

    al2 = attn_l2.reshape(OUT_FEATS, 1)
    ar2 = attn_r2.reshape(OUT_FEATS, 1)
    zp2 = jnp.zeros((H1 * HID, _HP - 1), f32)
    wbig2 = jnp.concatenate(
        [W2[:, :32], W2 @ al2, zp2,
         W2[:, 32:], W2 @ al2, zp2,
         W2 @ ar2, zp2, W2 @ ar2, zp2], axis=1)

    ex = jnp.repeat(jnp.eye(H1, dtype=f32), HID, axis=1)
    e0 = jnp.concatenate([ex[:hh], jnp.zeros((_HP - hh, IN_FEATS), f32)], 0)
    e1 = jnp.concatenate([ex[hh:], jnp.zeros((_HP - hh, IN_FEATS), f32)], 0)

    fel0, fel1, er0, er1 = _dense(x, wbig1, 64 + _HP)

    p1 = _make_edge_sc(64, H1 // 2, 128)(fel0, fel1, er0, er1, src, dst)

    f2a, f2b, er2a, er2b = _combine1(p1[0], p1[1], e0, e1,
                                     b1.reshape(1, -1), wbig2, 32 + _HP)

    p2 = _make_edge_sc(32, 1, 128)(f2a, f2b, er2a, er2b, src, dst)

    return _combine2(p2[0], p2[1], b2.reshape(1, -1))


# speedup vs baseline: 2.2710x; 2.2710x over previous
import functools

import jax
import jax.numpy as jnp
from jax import lax
from jax.experimental import pallas as pl
from jax.experimental.pallas import tpu as pltpu
from jax.experimental.pallas import tpu_sc as plsc

N = 10000
E = 320000
IN_FEATS = 128
HID = 16
OUT_FEATS = 64
H1 = 8

_BLK = 400

_B = 64
_NSUB = 16
_HP = 16


def _dense_body(x_ref, w_ref, o1_ref, o2_ref):
    z = jnp.dot(x_ref[...], w_ref[...], preferred_element_type=jnp.float32)
    cfel = o1_ref.shape[1]
    o1_ref[...] = z[:, :cfel]
    o2_ref[...] = z[:, cfel:]


def _dense(x, wbig, cfel):
    n, k = x.shape
    m = wbig.shape[1]
    return pl.pallas_call(
        _dense_body,
        grid=(n // _BLK,),
        in_specs=[
            pl.BlockSpec((_BLK, k), lambda i: (i, 0)),
            pl.BlockSpec((k, m), lambda i: (0, 0)),
        ],
        out_specs=[
            pl.BlockSpec((_BLK, cfel), lambda i: (i, 0)),
            pl.BlockSpec((_BLK, m - cfel), lambda i: (i, 0)),
        ],
        out_shape=[
            jax.ShapeDtypeStruct((n, cfel), jnp.float32),
            jax.ShapeDtypeStruct((n, m - cfel), jnp.float32),
        ],
    )(x, wbig)


def _combine1_body(pa_ref, pb_ref, exp_ref, b_ref, w_ref, o1_ref, o2_ref):
    pa = pa_ref[...] + pb_ref[...]
    num = pa[:, :IN_FEATS]
    den = pa[:, IN_FEATS:]
    denx = jnp.dot(den, exp_ref[...], preferred_element_type=jnp.float32)
    h = num / (denx + 1e-9) + b_ref[...]
    h = jnp.maximum(h, 0.0)
    z = jnp.dot(h, w_ref[...], preferred_element_type=jnp.float32)
    cfel = o1_ref.shape[1]
    o1_ref[...] = z[:, :cfel]
    o2_ref[...] = z[:, cfel:]


def _combine1(pa, pb, expand, b1, w2big, cfel):
    n = pa.shape[0]
    cc = pa.shape[1]
    m = w2big.shape[1]
    blk = pl.BlockSpec((_BLK, cc), lambda i: (i, 0))
    return pl.pallas_call(
        _combine1_body,
        grid=(n // _BLK,),
        in_specs=[
            blk, blk,
            pl.BlockSpec((_HP, IN_FEATS), lambda i: (0, 0)),
            pl.BlockSpec((1, IN_FEATS), lambda i: (0, 0)),
            pl.BlockSpec((IN_FEATS, m), lambda i: (0, 0)),
        ],
        out_specs=[
            pl.BlockSpec((_BLK, cfel), lambda i: (i, 0)),
            pl.BlockSpec((_BLK, m - cfel), lambda i: (i, 0)),
        ],
        out_shape=[
            jax.ShapeDtypeStruct((n, cfel), jnp.float32),
            jax.ShapeDtypeStruct((n, m - cfel), jnp.float32),
        ],
    )(pa, pb, expand, b1, w2big)


def _combine2_body(pa_ref, pb_ref, b_ref, o_ref):
    pa = pa_ref[...] + pb_ref[...]
    num = pa[:, :OUT_FEATS]
    den = pa[:, OUT_FEATS:OUT_FEATS + 1]
    o_ref[...] = num / (den + 1e-9) + b_ref[...]


def _combine2(pa, pb, b2):
    n = pa.shape[0]
    cc = pa.shape[1]
    blk = pl.BlockSpec((_BLK, cc), lambda i: (i, 0))
    return pl.pallas_call(
        _combine2_body,
        grid=(n // _BLK,),
        in_specs=[
            blk, blk,
            pl.BlockSpec((1, OUT_FEATS), lambda i: (0, 0)),
        ],
        out_specs=pl.BlockSpec((_BLK, OUT_FEATS), lambda i: (i, 0)),
        out_shape=jax.ShapeDtypeStruct((n, OUT_FEATS), jnp.float32),
    )(pa, pb, b2)


@functools.lru_cache(maxsize=None)
def _make_edge_sc(C, H):
    D = C // H
    CC = C + _HP
    NB = E // _B
    NBC = NB // 2
    NBT = NBC // _NSUB
    NEXTRA = NBC % _NSUB
    CNT_MAX = NBT + (1 if NEXTRA else 0)
    ROWS_T = (N // _NSUB) // 8 * 8
    ROWS_REM = N - ROWS_T * _NSUB

    mesh = plsc.VectorSubcoreMesh(core_axis_name="c", subcore_axis_name="s")

    def buf_pair(shape, dtype):
        return [pltpu.MemorySpace.VMEM(shape, dtype) for _ in range(2)]

    @functools.partial(
        pl.kernel,
        out_type=jax.ShapeDtypeStruct((2, N, CC), jnp.float32),
        mesh=mesh,
        compiler_params=pltpu.CompilerParams(use_tc_tiling_on_sc=False),
        scratch_types=[
            pltpu.MemorySpace.VMEM_SHARED((N, CC), jnp.float32),
            buf_pair((_B,), jnp.int32),
            buf_pair((1, _B), jnp.int32),
            buf_pair((1, _B), jnp.int32),
            buf_pair((_B, _HP), jnp.float32),
            buf_pair((_B, CC), jnp.float32),
            buf_pair((_B, CC), jnp.float32),
            [pltpu.SemaphoreType.DMA for _ in range(2)],
            [pltpu.SemaphoreType.DMA for _ in range(2)],
            [pltpu.SemaphoreType.DMA for _ in range(2)],
        ],
    )
    def edge_kernel(fel, er, src, dst, acc_out,
                    acc, sidx, didx, didxs, erb, fbg, mb, isem, gsem, ssem):
        c = lax.axis_index("c")
        s = lax.axis_index("s")
        zeros16 = jnp.zeros((16,), jnp.float32)

        nmine = NBT + jnp.where(s < NEXTRA, 1, 0) if NEXTRA else NBT
        start_blk = c * NBC + NBT * s + jnp.minimum(s, NEXTRA)

        def issue_idx(b, k):
            off = (start_blk + k) * _B
            pltpu.async_copy(src.at[pl.ds(off, _B)], sidx[b], isem[b])
            pltpu.async_copy(dst.at[pl.ds(off, _B)], didx[b].at[0], isem[b])

        def wait_idx(b, k):
            off = (start_blk + k) * _B
            pltpu.make_async_copy(src.at[pl.ds(off, _B)], sidx[b],
                                  isem[b]).wait()
            pltpu.make_async_copy(dst.at[pl.ds(off, _B)], didx[b].at[0],
                                  isem[b]).wait()

        def issue_gathers(b):
            pltpu.async_copy(fel.at[sidx[b]], fbg[b], gsem[b])
            pltpu.async_copy(er.at[didx[b].at[0]], erb[b], gsem[b])

        def wait_gathers(b):
            pltpu.make_async_copy(fel.at[sidx[b]], fbg[b], gsem[b]).wait()
            pltpu.make_async_copy(er.at[didx[b].at[0]], erb[b], gsem[b]).wait()

        def issue_scatter(b):
            pltpu.async_copy(mb[b], acc.at[didxs[b].at[0]], ssem[b], add=True)

        def wait_scatter(b):
            pltpu.make_async_copy(mb[b], acc.at[didxs[b].at[0]],
                                  ssem[b]).wait()

        def compute(b):
            @pl.loop(0, _B)
            def _m(i):
                ev = fbg[b][i, pl.ds(C, 16)] + erb[b][i, :]
                wrow = jnp.exp(jnp.maximum(ev, 0.2 * ev))
                mb[b][i, pl.ds(C, 16)] = wrow
                for h in range(H):
                    wv = jnp.broadcast_to(wrow[h], (16,))
                    for t in range(D // 16):
                        col = h * D + t * 16
                        mb[b][i, pl.ds(col, 16)] = fbg[b][i, pl.ds(col, 16)] * wv

        for b in range(2):
            @pl.when(b < nmine)
            def _prime():
                issue_idx(b, b)
                wait_idx(b, b)
                issue_gathers(b)

        @pl.loop(0, _B * CC // 16)
        def _zf(k):
            i = k // (CC // 16)
            j = k % (CC // 16)
            mb[0][i, pl.ds(j * 16, 16)] = zeros16

        r0 = s * ROWS_T
        nfull = ROWS_T // _B
        rem = ROWS_T - nfull * _B
        for q in range(nfull):
            pltpu.sync_copy(mb[0], acc.at[pl.ds(r0 + q * _B, _B)])
        if rem:
            pltpu.sync_copy(mb[0].at[pl.ds(0, rem)],
                            acc.at[pl.ds(r0 + nfull * _B, rem)])

        @pl.when(s == _NSUB - 1)
        def _ztail():
            base = ROWS_T * _NSUB
            pltpu.sync_copy(mb[0].at[pl.ds(0, ROWS_REM)],
                            acc.at[pl.ds(base, ROWS_REM)])

        plsc.subcore_barrier()

        @pl.loop(0, (CNT_MAX + 1) // 2)
        def _blk(p):
            for b in range(2):
                k = p * 2 + b

                @pl.when(k < nmine)
                def _do():
                    wait_gathers(b)

                    @pl.loop(0, _B // 16)
                    def _ci(q):
                        didxs[b][0, pl.ds(q * 16, 16)] = \
                            didx[b][0, pl.ds(q * 16, 16)]

                    @pl.when(k + 2 < nmine)
                    def _ni():
                        issue_idx(b, k + 2)

                    @pl.when(k >= 2)
                    def _ws():
                        wait_scatter(b)

                    compute(b)
                    issue_scatter(b)

                    @pl.when(k + 2 < nmine)
                    def _nx():
                        wait_idx(b, k + 2)
                        issue_gathers(b)

        for b in range(2):
            @pl.when(b < nmine)
            def _drain():
                wait_scatter(b)

        plsc.subcore_barrier()
        pltpu.sync_copy(acc.at[pl.ds(r0, ROWS_T)],
                        acc_out.at[c, pl.ds(r0, ROWS_T)])

        @pl.when(s == _NSUB - 1)
        def _ftail():
            base = ROWS_T * _NSUB
            pltpu.sync_copy(acc.at[pl.ds(base, ROWS_REM)],
                            acc_out.at[c, pl.ds(base, ROWS_REM)])

    return edge_kernel


def kernel(x, edge_index, W1, attn_l1, attn_r1, b1, W2, attn_l2, attn_r2, b2):
    src = edge_index[0]
    dst = edge_index[1]

    al1 = jax.scipy.linalg.block_diag(
        *[attn_l1[h][:, None] for h in range(H1)])
    ar1 = jax.scipy.linalg.block_diag(
        *[attn_r1[h][:, None] for h in range(H1)])
    zp1 = jnp.zeros((IN_FEATS, _HP - H1), jnp.float32)
    wbig1 = jnp.concatenate([W1, W1 @ al1, zp1, W1 @ ar1, zp1],
                            axis=1)

    al2 = attn_l2.reshape(OUT_FEATS, 1)
    ar2 = attn_r2.reshape(OUT_FEATS, 1)
    zp2 = jnp.zeros((H1 * HID, _HP - 1), jnp.float32)
    wbig2 = jnp.concatenate([W2, W2 @ al2, zp2, W2 @ ar2, zp2],
                            axis=1)

    expand1 = jnp.concatenate(
        [jnp.repeat(jnp.eye(H1, dtype=jnp.float32), HID, axis=1),
         jnp.zeros((_HP - H1, H1 * HID), jnp.float32)], axis=0)

    fel1, er1 = _dense(x, wbig1, IN_FEATS + _HP)

    p1 = _make_edge_sc(IN_FEATS, H1)(fel1, er1, src, dst)

    fel2, er2 = _combine1(p1[0], p1[1], expand1, b1.reshape(1, -1), wbig2,
                          OUT_FEATS + _HP)

    p2 = _make_edge_sc(OUT_FEATS, 1)(fel2, er2, src, dst)

    return _combine2(p2[0], p2[1], b2.reshape(1, -1))
